# Initial kernel scaffold; baseline (speedup 1.0000x reference)
import jax
import jax.numpy as jnp
from jax.experimental import pallas as pl

BS = 1024
NJ = 17
TP = 24
TOKN = 34
TOKP = 40
TCN = 2048
TD = 512
NBLK = 4

ENC_B = 32
VQ_B = 32
DEC_B = 16

_F32 = jnp.float32


def _full_spec(x):
    nd = x.ndim
    return pl.BlockSpec(x.shape, lambda i, _nd=nd: (0,) * _nd)


def _ln(x, g, b):
    mu = jnp.mean(x, axis=-1, keepdims=True)
    var = jnp.mean((x - mu) ** 2, axis=-1, keepdims=True)
    return (x - mu) * jax.lax.rsqrt(var + 1e-5) * g + b


def _gelu(x):
    return jax.nn.gelu(x, approximate=False)


def _dot2d(a, b):
    return jax.lax.dot_general(a, b, (((1,), (0,)), ((), ())),
                               preferred_element_type=_F32)


def _tok_dot(w, x):
    bsz = x.shape[0]
    wb = jnp.broadcast_to(w[None], (bsz,) + w.shape)
    return jax.lax.dot_general(wb, x, (((1,), (1,)), ((0,), (0,))),
                               preferred_element_type=_F32)


def _ch_dot(x3, w):
    b, t, d = x3.shape
    y2 = _dot2d(x3.reshape(b * t, d), w)
    return y2.reshape(b, t, w.shape[1])


def _mixer(x, p, i):
    (l1g, l1b, tw1, tb1, tw2, tb2, l2g, l2b, cw1, cb1, cw2, cb2) = p
    y = _ln(x, l1g[i], l1b[i])
    t = _tok_dot(tw1[i], y) + tb1[i][None, :, None]
    t = _gelu(t)
    y = _tok_dot(tw2[i], t) + tb2[i][None, :, None]
    z = _ln(x + y, l2g[i], l2b[i])
    z = _ch_dot(z, cw1[i]) + cb1[i][None, None, :]
    z = _gelu(z)
    z = _ch_dot(z, cw2[i]) + cb2[i][None, None, :]
    return x + y + z


def _enc_body(j_ref, inv_ref, sw_ref, sb_ref,
              l1g, l1b, tw1, tb1, tw2, tb2, l2g, l2b, cw1, cb1, cw2, cb2,
              elg, elb, tmw, tmb, few, feb, out_ref):
    j = j_ref[...]
    sw = sw_ref[...]
    x = (j[:, :, 0:1] * sw[0][None, None, :]
         + j[:, :, 1:2] * sw[1][None, None, :]
         + sb_ref[...][None, None, :])
    vis = j[:, :, 2:3] != 0.0
    x = jnp.where(vis, x, inv_ref[...])
    p = (l1g[...], l1b[...], tw1[...], tb1[...], tw2[...], tb2[...],
         l2g[...], l2b[...], cw1[...], cb1[...], cw2[...], cb2[...])
    for i in range(NBLK):
        x = _mixer(x, p, i)
    x = _ln(x, elg[...], elb[...])
    x = _tok_dot(tmw[...], x) + tmb[...][None, :, None]
    x = _ch_dot(x, few[...]) + feb[...][None, None, :]
    out_ref[...] = x


def _vq_body(f_ref, cb_ref, out_ref):
    f = f_ref[...].reshape(-1, TD)
    cb = cb_ref[...]
    d = jax.lax.dot_general(f, cb, (((1,), (1,)), ((), ())),
                            preferred_element_type=_F32)
    cn = jnp.sum(cb * cb, axis=1)
    dist = cn[None, :] - 2.0 * d
    idx = jnp.argmin(dist, axis=1).astype(jnp.int32)
    out_ref[...] = idx.reshape(VQ_B, TOKP)


def _dec_body(cls_ref, cb_ref, mk_ref, dtb_ref, dsw_ref, dsb_ref,
              l1g, l1b, tw1, tb1, tw2, tb2, l2g, l2b, cw1, cb1, cw2, cb2,
              dlg, dlb, rw_ref, rb_ref, out_ref):
    part = _dot2d(cls_ref[...], cb_ref[...])
    pt = _dot2d(mk_ref[...], part)
    x = pt.reshape(DEC_B, TP, TD) + dtb_ref[...][None, :, None]
    x = _ch_dot(x, dsw_ref[...]) + dsb_ref[...][None, None, :]
    p = (l1g[...], l1b[...], tw1[...], tb1[...], tw2[...], tb2[...],
         l2g[...], l2b[...], cw1[...], cb1[...], cw2[...], cb2[...])
    for i in range(NBLK):
        x = _mixer(x, p, i)
    x = _ln(x, dlg[...], dlb[...])
    r = _ch_dot(x, rw_ref[...]) + rb_ref[...][None, None, :]
    out_ref[...] = r


def _pad_axis(a, axis, n):
    pads = [(0, 0)] * a.ndim
    pads[axis] = (0, n - a.shape[axis])
    return jnp.pad(a, pads)


def kernel(joints, joints_feature, cls_logits, params):
    del joints_feature
    p = params
    enc, dec = p['enc'], p['dec']

    jp = _pad_axis(joints.astype(_F32), 1, TP)

    enc_args = [
        jp, p['invisible_token'], p['start_W'], p['start_b'],
        enc['ln1_g'], enc['ln1_b'],
        _pad_axis(enc['tok_W1'], 1, TP), enc['tok_b1'],
        _pad_axis(enc['tok_W2'], 2, TP), _pad_axis(enc['tok_b2'], 1, TP),
        enc['ln2_g'], enc['ln2_b'],
        enc['ch_W1'], enc['ch_b1'], enc['ch_W2'], enc['ch_b2'],
        p['enc_ln_g'], p['enc_ln_b'],
        _pad_axis(_pad_axis(p['tokmlp_W'], 0, TP), 1, TOKP),
        _pad_axis(p['tokmlp_b'], 0, TOKP),
        p['featemb_W'], p['featemb_b'],
    ]
    enc_specs = [pl.BlockSpec((ENC_B, TP, 3), lambda i: (i, 0, 0))]
    enc_specs += [_full_spec(a) for a in enc_args[1:]]
    feat = pl.pallas_call(
        _enc_body,
        grid=(BS // ENC_B,),
        in_specs=enc_specs,
        out_specs=pl.BlockSpec((ENC_B, TOKP, TD), lambda i: (i, 0, 0)),
        out_shape=jax.ShapeDtypeStruct((BS, TOKP, TD), _F32),
    )(*enc_args)

    cb = p['codebook']
    idx_pad = pl.pallas_call(
        _vq_body,
        grid=(BS // VQ_B,),
        in_specs=[pl.BlockSpec((VQ_B, TOKP, TD), lambda i: (i, 0, 0)),
                  _full_spec(cb)],
        out_specs=pl.BlockSpec((VQ_B, TOKP), lambda i: (i, 0)),
        out_shape=jax.ShapeDtypeStruct((BS, TOKP), jnp.int32),
    )(feat, cb)
    idx = idx_pad[:, :TOKN].reshape(-1)

    wp = _pad_axis(p['dectok_W'], 1, TP)
    eye = jnp.eye(DEC_B, dtype=_F32)
    mk = (eye[:, None, :, None] * wp.T[None, :, None, :]).reshape(
        DEC_B * TP, DEC_B * TOKN)

    dec_args = [
        cls_logits, cb, mk, _pad_axis(p['dectok_b'], 0, TP),
        p['decstart_W'], p['decstart_b'],
        dec['ln1_g'], dec['ln1_b'],
        _pad_axis(dec['tok_W1'], 1, TP), dec['tok_b1'],
        _pad_axis(dec['tok_W2'], 2, TP), _pad_axis(dec['tok_b2'], 1, TP),
        dec['ln2_g'], dec['ln2_b'],
        dec['ch_W1'], dec['ch_b1'], dec['ch_W2'], dec['ch_b2'],
        p['dec_ln_g'], p['dec_ln_b'],
        p['rec_W'], p['rec_b'],
    ]
    dec_specs = [pl.BlockSpec((DEC_B * TOKN, TCN), lambda i: (i, 0))]
    dec_specs += [_full_spec(a) for a in dec_args[1:]]
    rec_pad = pl.pallas_call(
        _dec_body,
        grid=(BS // DEC_B,),
        in_specs=dec_specs,
        out_specs=pl.BlockSpec((DEC_B, TP, 2), lambda i: (i, 0, 0)),
        out_shape=jax.ShapeDtypeStruct((BS, TP, 2), _F32),
    )(*dec_args)
    rec = rec_pad[:, :NJ, :]
    return rec, idx


# speedup vs baseline: 3.7777x; 3.7777x over previous
import jax
import jax.numpy as jnp
from jax.experimental import pallas as pl

BS = 1024
NJ = 17
TP = 24
TOKN = 34
TOKP = 40
TCN = 2048
TD = 512
NBLK = 4

ENC_B = 32
VQR = 1024
DEC_B = 16

_F32 = jnp.float32


def _full_spec(x):
    nd = x.ndim
    return pl.BlockSpec(x.shape, lambda i, _nd=nd: (0,) * _nd)


def _ln(x, g, b):
    mu = jnp.mean(x, axis=-1, keepdims=True)
    var = jnp.mean((x - mu) ** 2, axis=-1, keepdims=True)
    return (x - mu) * jax.lax.rsqrt(var + 1e-5) * g + b


def _gelu(x):
    return 0.5 * x * (1.0 + jax.lax.erf(x * (2.0 ** -0.5)))


def _dot2d(a, b):
    return jax.lax.dot_general(a, b, (((1,), (0,)), ((), ())),
                               preferred_element_type=_F32)


def _tok_dot(w, x):
    bsz = x.shape[0]
    wb = jnp.broadcast_to(w[None], (bsz,) + w.shape)
    return jax.lax.dot_general(wb, x, (((1,), (1,)), ((0,), (0,))),
                               preferred_element_type=_F32)


def _ch_dot(x3, w):
    b, t, d = x3.shape
    y2 = _dot2d(x3.reshape(b * t, d), w)
    return y2.reshape(b, t, w.shape[1])


def _mixer(x, p, i):
    (l1g, l1b, tw1, tb1, tw2, tb2, l2g, l2b, cw1, cb1, cw2, cb2) = p
    y = _ln(x, l1g[i], l1b[i])
    t = _tok_dot(tw1[i], y) + tb1[i][None, :, None]
    t = _gelu(t)
    y = _tok_dot(tw2[i], t) + tb2[i][None, :, None]
    z = _ln(x + y, l2g[i], l2b[i])
    z = _ch_dot(z, cw1[i]) + cb1[i][None, None, :]
    z = _gelu(z)
    z = _ch_dot(z, cw2[i]) + cb2[i][None, None, :]
    return x + y + z


def _enc_body(j_ref, inv_ref, sw_ref, sb_ref,
              l1g, l1b, tw1, tb1, tw2, tb2, l2g, l2b, cw1, cb1, cw2, cb2,
              elg, elb, tmw, tmb, few, feb, out_ref):
    j = j_ref[...]
    sw = sw_ref[...]
    x = (j[:, :, 0:1] * sw[0][None, None, :]
         + j[:, :, 1:2] * sw[1][None, None, :]
         + sb_ref[...][None, None, :])
    vis = j[:, :, 2:3] != 0.0
    x = jnp.where(vis, x, inv_ref[...])
    p = (l1g[...], l1b[...], tw1[...], tb1[...], tw2[...], tb2[...],
         l2g[...], l2b[...], cw1[...], cb1[...], cw2[...], cb2[...])
    for i in range(NBLK):
        x = _mixer(x, p, i)
    x = _ln(x, elg[...], elb[...])
    x = _tok_dot(tmw[...], x) + tmb[...][None, :, None]
    x = _ch_dot(x, few[...]) + feb[...][None, None, :]
    out_ref[...] = x


def _vq_body(f_ref, cb_ref, out_ref):
    f = f_ref[...]
    cb = cb_ref[...]
    d = jax.lax.dot_general(f, cb, (((1,), (1,)), ((), ())),
                            preferred_element_type=_F32)
    ones = jnp.ones((8, TD), dtype=_F32)
    cn = jax.lax.dot_general(ones, cb * cb, (((1,), (1,)), ((), ())),
                             preferred_element_type=_F32)
    dist = cn[0:1, :] - 2.0 * d
    m = jnp.min(dist, axis=1, keepdims=True)
    it = jax.lax.broadcasted_iota(jnp.int32, dist.shape, 1)
    idx = jnp.min(jnp.where(dist == m, it, TCN), axis=1)
    out_ref[...] = idx.reshape(VQR // 128, 128)


def _dec_body(cls_ref, cb_ref, mk_ref, dtb_ref, dsw_ref, dsb_ref,
              l1g, l1b, tw1, tb1, tw2, tb2, l2g, l2b, cw1, cb1, cw2, cb2,
              dlg, dlb, rw_ref, rb_ref, out_ref):
    part = _dot2d(cls_ref[...], cb_ref[...])
    pt = _dot2d(mk_ref[...], part)
    x = pt.reshape(DEC_B, TP, TD) + dtb_ref[...][None, :, None]
    x = _ch_dot(x, dsw_ref[...]) + dsb_ref[...][None, None, :]
    p = (l1g[...], l1b[...], tw1[...], tb1[...], tw2[...], tb2[...],
         l2g[...], l2b[...], cw1[...], cb1[...], cw2[...], cb2[...])
    for i in range(NBLK):
        x = _mixer(x, p, i)
    x = _ln(x, dlg[...], dlb[...])
    r = _ch_dot(x, rw_ref[...]) + rb_ref[...][None, None, :]
    out_ref[...] = r


def _pad_axis(a, axis, n):
    pads = [(0, 0)] * a.ndim
    pads[axis] = (0, n - a.shape[axis])
    return jnp.pad(a, pads)


def kernel(joints, joints_feature, cls_logits, params):
    del joints_feature
    p = params
    enc, dec = p['enc'], p['dec']

    jp = _pad_axis(joints.astype(_F32), 1, TP)

    enc_args = [
        jp, p['invisible_token'], p['start_W'], p['start_b'],
        enc['ln1_g'], enc['ln1_b'],
        _pad_axis(enc['tok_W1'], 1, TP), enc['tok_b1'],
        _pad_axis(enc['tok_W2'], 2, TP), _pad_axis(enc['tok_b2'], 1, TP),
        enc['ln2_g'], enc['ln2_b'],
        enc['ch_W1'], enc['ch_b1'], enc['ch_W2'], enc['ch_b2'],
        p['enc_ln_g'], p['enc_ln_b'],
        _pad_axis(_pad_axis(p['tokmlp_W'], 0, TP), 1, TOKP),
        _pad_axis(p['tokmlp_b'], 0, TOKP),
        p['featemb_W'], p['featemb_b'],
    ]
    enc_specs = [pl.BlockSpec((ENC_B, TP, 3), lambda i: (i, 0, 0))]
    enc_specs += [_full_spec(a) for a in enc_args[1:]]
    feat = pl.pallas_call(
        _enc_body,
        grid=(BS // ENC_B,),
        in_specs=enc_specs,
        out_specs=pl.BlockSpec((ENC_B, TOKP, TD), lambda i: (i, 0, 0)),
        out_shape=jax.ShapeDtypeStruct((BS, TOKP, TD), _F32),
    )(*enc_args)

    cb = p['codebook']
    nrow = BS * TOKP
    idx_pad = pl.pallas_call(
        _vq_body,
        grid=(nrow // VQR,),
        in_specs=[pl.BlockSpec((VQR, TD), lambda i: (i, 0)),
                  _full_spec(cb)],
        out_specs=pl.BlockSpec((VQR // 128, 128), lambda i: (i, 0)),
        out_shape=jax.ShapeDtypeStruct((nrow // 128, 128), jnp.int32),
    )(feat.reshape(nrow, TD), cb)
    idx = idx_pad.reshape(BS, TOKP)[:, :TOKN].reshape(-1)

    wp = _pad_axis(p['dectok_W'], 1, TP)
    eye = jnp.eye(DEC_B, dtype=_F32)
    mk = (eye[:, None, :, None] * wp.T[None, :, None, :]).reshape(
        DEC_B * TP, DEC_B * TOKN)

    dec_args = [
        cls_logits, cb, mk, _pad_axis(p['dectok_b'], 0, TP),
        p['decstart_W'], p['decstart_b'],
        dec['ln1_g'], dec['ln1_b'],
        _pad_axis(dec['tok_W1'], 1, TP), dec['tok_b1'],
        _pad_axis(dec['tok_W2'], 2, TP), _pad_axis(dec['tok_b2'], 1, TP),
        dec['ln2_g'], dec['ln2_b'],
        dec['ch_W1'], dec['ch_b1'], dec['ch_W2'], dec['ch_b2'],
        p['dec_ln_g'], p['dec_ln_b'],
        p['rec_W'], p['rec_b'],
    ]
    dec_specs = [pl.BlockSpec((DEC_B * TOKN, TCN), lambda i: (i, 0))]
    dec_specs += [_full_spec(a) for a in dec_args[1:]]
    rec_pad = pl.pallas_call(
        _dec_body,
        grid=(BS // DEC_B,),
        in_specs=dec_specs,
        out_specs=pl.BlockSpec((DEC_B, TP, 2), lambda i: (i, 0, 0)),
        out_shape=jax.ShapeDtypeStruct((BS, TP, 2), _F32),
    )(*dec_args)
    rec = rec_pad[:, :NJ, :]
    return rec, idx


# speedup vs baseline: 4.2645x; 1.1289x over previous
import jax
import jax.numpy as jnp
from jax.experimental import pallas as pl

BS = 1024
NJ = 17
TP = 24
TOKN = 34
TOKP = 40
TCN = 2048
TD = 512
NBLK = 4

ENC_B = 32
VQR = 1024
DEC_B = 32

_F32 = jnp.float32


def _full_spec(x):
    nd = x.ndim
    return pl.BlockSpec(x.shape, lambda i, _nd=nd: (0,) * _nd)


def _ln(x, g, b):
    mu = jnp.mean(x, axis=-1, keepdims=True)
    m2 = jnp.mean(x * x, axis=-1, keepdims=True)
    var = m2 - mu * mu
    return (x - mu) * jax.lax.rsqrt(var + 1e-5) * g + b


def _gelu(x):
    return 0.5 * x * (1.0 + jax.lax.erf(x * (2.0 ** -0.5)))


def _dot2d(a, b):
    return jax.lax.dot_general(a, b, (((1,), (0,)), ((), ())),
                               preferred_element_type=_F32)


def _tok_dot(w, x):
    bsz = x.shape[0]
    wb = jnp.broadcast_to(w[None], (bsz,) + w.shape)
    return jax.lax.dot_general(wb, x, (((1,), (1,)), ((0,), (0,))),
                               preferred_element_type=_F32)


def _ch_dot(x3, w):
    b, t, d = x3.shape
    y2 = _dot2d(x3.reshape(b * t, d), w)
    return y2.reshape(b, t, w.shape[1])


def _mixer(x, p, i):
    (l1g, l1b, tw1, tb1, tw2, tb2, l2g, l2b, cw1, cb1, cw2, cb2) = p
    y = _ln(x, l1g[i], l1b[i])
    t = _tok_dot(tw1[i], y) + tb1[i][None, :, None]
    t = _gelu(t)
    y = _tok_dot(tw2[i], t) + tb2[i][None, :, None]
    z = _ln(x + y, l2g[i], l2b[i])
    z = _ch_dot(z, cw1[i]) + cb1[i][None, None, :]
    z = _gelu(z)
    z = _ch_dot(z, cw2[i]) + cb2[i][None, None, :]
    return x + y + z


def _enc_body(j_ref, inv_ref, sw_ref, sb_ref,
              l1g, l1b, tw1, tb1, tw2, tb2, l2g, l2b, cw1, cb1, cw2, cb2,
              elg, elb, tmw, tmb, few, feb, out_ref):
    j = j_ref[...]
    sw = sw_ref[...]
    x = (j[:, :, 0:1] * sw[0][None, None, :]
         + j[:, :, 1:2] * sw[1][None, None, :]
         + sb_ref[...][None, None, :])
    vis = j[:, :, 2:3] != 0.0
    x = jnp.where(vis, x, inv_ref[...])
    p = (l1g[...], l1b[...], tw1[...], tb1[...], tw2[...], tb2[...],
         l2g[...], l2b[...], cw1[...], cb1[...], cw2[...], cb2[...])
    for i in range(NBLK):
        x = _mixer(x, p, i)
    x = _ln(x, elg[...], elb[...])
    x = _tok_dot(tmw[...], x) + tmb[...][None, :, None]
    x = _ch_dot(x, few[...]) + feb[...][None, None, :]
    out_ref[...] = x


def _vq_body(f_ref, cb_ref, out_ref):
    f = f_ref[...]
    cb = cb_ref[...]
    d = jax.lax.dot_general(f, cb, (((1,), (1,)), ((), ())),
                            preferred_element_type=_F32)
    ones = jnp.ones((8, TD), dtype=_F32)
    cn = jax.lax.dot_general(ones, cb * cb, (((1,), (1,)), ((), ())),
                             preferred_element_type=_F32)
    dist = cn[0:1, :] - 2.0 * d
    idx = jnp.argmin(dist, axis=1).astype(jnp.int32)
    out_ref[...] = idx.reshape(VQR // 128, 128)


def _dec_body(cls_ref, cb_ref, mk_ref, dtb_ref, dsw_ref, dsb_ref,
              l1g, l1b, tw1, tb1, tw2, tb2, l2g, l2b, cw1, cb1, cw2, cb2,
              dlg, dlb, rw_ref, rb_ref, out_ref):
    part = jax.lax.dot_general(
        cls_ref[...].astype(jnp.bfloat16), cb_ref[...].astype(jnp.bfloat16),
        (((1,), (0,)), ((), ())),
        preferred_element_type=_F32)
    pt = _dot2d(mk_ref[...], part)
    x = pt.reshape(DEC_B, TP, TD) + dtb_ref[...][None, :, None]
    x = _ch_dot(x, dsw_ref[...]) + dsb_ref[...][None, None, :]
    p = (l1g[...], l1b[...], tw1[...], tb1[...], tw2[...], tb2[...],
         l2g[...], l2b[...], cw1[...], cb1[...], cw2[...], cb2[...])
    for i in range(NBLK):
        x = _mixer(x, p, i)
    x = _ln(x, dlg[...], dlb[...])
    r = _ch_dot(x, rw_ref[...]) + rb_ref[...][None, None, :]
    out_ref[...] = r


def _pad_axis(a, axis, n):
    pads = [(0, 0)] * a.ndim
    pads[axis] = (0, n - a.shape[axis])
    return jnp.pad(a, pads)


def kernel(joints, joints_feature, cls_logits, params):
    del joints_feature
    p = params
    enc, dec = p['enc'], p['dec']

    jp = _pad_axis(joints.astype(_F32), 1, TP)

    enc_args = [
        jp, p['invisible_token'], p['start_W'], p['start_b'],
        enc['ln1_g'], enc['ln1_b'],
        _pad_axis(enc['tok_W1'], 1, TP), enc['tok_b1'],
        _pad_axis(enc['tok_W2'], 2, TP), _pad_axis(enc['tok_b2'], 1, TP),
        enc['ln2_g'], enc['ln2_b'],
        enc['ch_W1'], enc['ch_b1'], enc['ch_W2'], enc['ch_b2'],
        p['enc_ln_g'], p['enc_ln_b'],
        _pad_axis(_pad_axis(p['tokmlp_W'], 0, TP), 1, TOKP),
        _pad_axis(p['tokmlp_b'], 0, TOKP),
        p['featemb_W'], p['featemb_b'],
    ]
    enc_specs = [pl.BlockSpec((ENC_B, TP, 3), lambda i: (i, 0, 0))]
    enc_specs += [_full_spec(a) for a in enc_args[1:]]
    feat = pl.pallas_call(
        _enc_body,
        grid=(BS // ENC_B,),
        in_specs=enc_specs,
        out_specs=pl.BlockSpec((ENC_B, TOKP, TD), lambda i: (i, 0, 0)),
        out_shape=jax.ShapeDtypeStruct((BS, TOKP, TD), _F32),
    )(*enc_args)

    cb = p['codebook']
    nrow = BS * TOKP
    idx_pad = pl.pallas_call(
        _vq_body,
        grid=(nrow // VQR,),
        in_specs=[pl.BlockSpec((VQR, TD), lambda i: (i, 0)),
                  _full_spec(cb)],
        out_specs=pl.BlockSpec((VQR // 128, 128), lambda i: (i, 0)),
        out_shape=jax.ShapeDtypeStruct((nrow // 128, 128), jnp.int32),
    )(feat.reshape(nrow, TD), cb)
    idx = idx_pad.reshape(BS, TOKP)[:, :TOKN].reshape(-1)

    wp = _pad_axis(p['dectok_W'], 1, TP)
    eye = jnp.eye(DEC_B, dtype=_F32)
    mk = (eye[:, None, :, None] * wp.T[None, :, None, :]).reshape(
        DEC_B * TP, DEC_B * TOKN)

    dec_args = [
        cls_logits, cb, mk, _pad_axis(p['dectok_b'], 0, TP),
        p['decstart_W'], p['decstart_b'],
        dec['ln1_g'], dec['ln1_b'],
        _pad_axis(dec['tok_W1'], 1, TP), dec['tok_b1'],
        _pad_axis(dec['tok_W2'], 2, TP), _pad_axis(dec['tok_b2'], 1, TP),
        dec['ln2_g'], dec['ln2_b'],
        dec['ch_W1'], dec['ch_b1'], dec['ch_W2'], dec['ch_b2'],
        p['dec_ln_g'], p['dec_ln_b'],
        p['rec_W'], p['rec_b'],
    ]
    dec_specs = [pl.BlockSpec((DEC_B * TOKN, TCN), lambda i: (i, 0))]
    dec_specs += [_full_spec(a) for a in dec_args[1:]]
    rec_pad = pl.pallas_call(
        _dec_body,
        grid=(BS // DEC_B,),
        in_specs=dec_specs,
        out_specs=pl.BlockSpec((DEC_B, TP, 2), lambda i: (i, 0, 0)),
        out_shape=jax.ShapeDtypeStruct((BS, TP, 2), _F32),
    )(*dec_args)
    rec = rec_pad[:, :NJ, :]
    return rec, idx


# speedup vs baseline: 4.2894x; 1.0059x over previous
import jax
import jax.numpy as jnp
from jax.experimental import pallas as pl

BS = 1024
NJ = 17
TP = 24
TOKN = 34
TOKP = 40
TCN = 2048
TD = 512
NBLK = 4

ENC_B = 32
VQR = 1024
DEC_B = 32

_F32 = jnp.float32


def _full_spec(x):
    nd = x.ndim
    return pl.BlockSpec(x.shape, lambda i, _nd=nd: (0,) * _nd)


def _ln(x, g, b):
    mu = jnp.mean(x, axis=-1, keepdims=True)
    m2 = jnp.mean(x * x, axis=-1, keepdims=True)
    var = m2 - mu * mu
    return (x - mu) * jax.lax.rsqrt(var + 1e-5) * g + b


def _gelu(x):
    return 0.5 * x * (1.0 + jax.lax.erf(x * (2.0 ** -0.5)))


def _dot2d(a, b):
    return jax.lax.dot_general(a, b, (((1,), (0,)), ((), ())),
                               preferred_element_type=_F32)


def _tok_dot(w, x):
    bsz = x.shape[0]
    wb = jnp.broadcast_to(w[None], (bsz,) + w.shape)
    return jax.lax.dot_general(wb, x, (((1,), (1,)), ((0,), (0,))),
                               preferred_element_type=_F32)


def _ch_dot(x3, w):
    b, t, d = x3.shape
    y2 = _dot2d(x3.reshape(b * t, d), w)
    return y2.reshape(b, t, w.shape[1])


def _mixer(x, p, i):
    (l1g, l1b, tw1, tb1, tw2, tb2, l2g, l2b, cw1, cb1, cw2, cb2) = p
    y = _ln(x, l1g[i], l1b[i])
    t = _tok_dot(tw1[i], y) + tb1[i][None, :, None]
    t = _gelu(t)
    y = _tok_dot(tw2[i], t) + tb2[i][None, :, None]
    z = _ln(x + y, l2g[i], l2b[i])
    z = _ch_dot(z, cw1[i]) + cb1[i][None, None, :]
    z = _gelu(z)
    z = _ch_dot(z, cw2[i]) + cb2[i][None, None, :]
    return x + y + z


def _enc_body(j_ref, inv_ref, sw_ref, sb_ref,
              l1g, l1b, tw1, tb1, tw2, tb2, l2g, l2b, cw1, cb1, cw2, cb2,
              elg, elb, tmw, tmb, few, feb, out_ref):
    j = j_ref[...]
    sw = sw_ref[...]
    x = (j[:, :, 0:1] * sw[0][None, None, :]
         + j[:, :, 1:2] * sw[1][None, None, :]
         + sb_ref[...][None, None, :])
    vis = j[:, :, 2:3] != 0.0
    x = jnp.where(vis, x, inv_ref[...])
    p = (l1g[...], l1b[...], tw1[...], tb1[...], tw2[...], tb2[...],
         l2g[...], l2b[...], cw1[...], cb1[...], cw2[...], cb2[...])
    for i in range(NBLK):
        x = _mixer(x, p, i)
    x = _ln(x, elg[...], elb[...])
    x = _tok_dot(tmw[...], x) + tmb[...][None, :, None]
    x = _ch_dot(x, few[...]) + feb[...][None, None, :]
    out_ref[...] = x


def _vq_body(f_ref, cb_ref, out_ref):
    f = f_ref[...]
    cb = cb_ref[...]
    d = jax.lax.dot_general(f, cb, (((1,), (1,)), ((), ())),
                            preferred_element_type=_F32)
    ones = jnp.ones((8, TD), dtype=_F32)
    cn = jax.lax.dot_general(ones, cb * cb, (((1,), (1,)), ((), ())),
                             preferred_element_type=_F32)
    dist = cn[0:1, :] - 2.0 * d
    m = jnp.min(dist, axis=1, keepdims=True)
    it = jax.lax.broadcasted_iota(jnp.int32, dist.shape, 1)
    idx = jnp.min(jnp.where(dist == m, it, TCN), axis=1)
    out_ref[...] = idx.reshape(VQR // 128, 128)


def _dec_body(cls_ref, cb_ref, mk_ref, dtb_ref, dsw_ref, dsb_ref,
              l1g, l1b, tw1, tb1, tw2, tb2, l2g, l2b, cw1, cb1, cw2, cb2,
              dlg, dlb, rw_ref, rb_ref, out_ref):
    part = jax.lax.dot_general(
        cls_ref[...].astype(jnp.bfloat16), cb_ref[...].astype(jnp.bfloat16),
        (((1,), (0,)), ((), ())),
        preferred_element_type=_F32)
    pt = _dot2d(mk_ref[...], part)
    x = pt.reshape(DEC_B, TP, TD) + dtb_ref[...][None, :, None]
    x = _ch_dot(x, dsw_ref[...]) + dsb_ref[...][None, None, :]
    p = (l1g[...], l1b[...], tw1[...], tb1[...], tw2[...], tb2[...],
         l2g[...], l2b[...], cw1[...], cb1[...], cw2[...], cb2[...])
    for i in range(NBLK):
        x = _mixer(x, p, i)
    x = _ln(x, dlg[...], dlb[...])
    r = _ch_dot(x, rw_ref[...]) + rb_ref[...][None, None, :]
    out_ref[...] = r


def _pad_axis(a, axis, n):
    pads = [(0, 0)] * a.ndim
    pads[axis] = (0, n - a.shape[axis])
    return jnp.pad(a, pads)


def kernel(joints, joints_feature, cls_logits, params):
    del joints_feature
    p = params
    enc, dec = p['enc'], p['dec']

    jp = _pad_axis(joints.astype(_F32), 1, TP)

    enc_args = [
        jp, p['invisible_token'], p['start_W'], p['start_b'],
        enc['ln1_g'], enc['ln1_b'],
        _pad_axis(enc['tok_W1'], 1, TP), enc['tok_b1'],
        _pad_axis(enc['tok_W2'], 2, TP), _pad_axis(enc['tok_b2'], 1, TP),
        enc['ln2_g'], enc['ln2_b'],
        enc['ch_W1'], enc['ch_b1'], enc['ch_W2'], enc['ch_b2'],
        p['enc_ln_g'], p['enc_ln_b'],
        _pad_axis(_pad_axis(p['tokmlp_W'], 0, TP), 1, TOKP),
        _pad_axis(p['tokmlp_b'], 0, TOKP),
        p['featemb_W'], p['featemb_b'],
    ]
    enc_specs = [pl.BlockSpec((ENC_B, TP, 3), lambda i: (i, 0, 0))]
    enc_specs += [_full_spec(a) for a in enc_args[1:]]
    feat = pl.pallas_call(
        _enc_body,
        grid=(BS // ENC_B,),
        in_specs=enc_specs,
        out_specs=pl.BlockSpec((ENC_B, TOKP, TD), lambda i: (i, 0, 0)),
        out_shape=jax.ShapeDtypeStruct((BS, TOKP, TD), _F32),
    )(*enc_args)

    cb = p['codebook']
    nrow = BS * TOKP
    idx_pad = pl.pallas_call(
        _vq_body,
        grid=(nrow // VQR,),
        in_specs=[pl.BlockSpec((VQR, TD), lambda i: (i, 0)),
                  _full_spec(cb)],
        out_specs=pl.BlockSpec((VQR // 128, 128), lambda i: (i, 0)),
        out_shape=jax.ShapeDtypeStruct((nrow // 128, 128), jnp.int32),
    )(feat.reshape(nrow, TD), cb)
    idx = idx_pad.reshape(BS, TOKP)[:, :TOKN].reshape(-1)

    wp = _pad_axis(p['dectok_W'], 1, TP)
    eye = jnp.eye(DEC_B, dtype=_F32)
    mk = (eye[:, None, :, None] * wp.T[None, :, None, :]).reshape(
        DEC_B * TP, DEC_B * TOKN)

    dec_args = [
        cls_logits, cb, mk, _pad_axis(p['dectok_b'], 0, TP),
        p['decstart_W'], p['decstart_b'],
        dec['ln1_g'], dec['ln1_b'],
        _pad_axis(dec['tok_W1'], 1, TP), dec['tok_b1'],
        _pad_axis(dec['tok_W2'], 2, TP), _pad_axis(dec['tok_b2'], 1, TP),
        dec['ln2_g'], dec['ln2_b'],
        dec['ch_W1'], dec['ch_b1'], dec['ch_W2'], dec['ch_b2'],
        p['dec_ln_g'], p['dec_ln_b'],
        p['rec_W'], p['rec_b'],
    ]
    dec_specs = [pl.BlockSpec((DEC_B * TOKN, TCN), lambda i: (i, 0))]
    dec_specs += [_full_spec(a) for a in dec_args[1:]]
    rec_pad = pl.pallas_call(
        _dec_body,
        grid=(BS // DEC_B,),
        in_specs=dec_specs,
        out_specs=pl.BlockSpec((DEC_B, TP, 2), lambda i: (i, 0, 0)),
        out_shape=jax.ShapeDtypeStruct((BS, TP, 2), _F32),
    )(*dec_args)
    rec = rec_pad[:, :NJ, :]
    return rec, idx


# speedup vs baseline: 4.3184x; 1.0067x over previous
import jax
import jax.numpy as jnp
from jax.experimental import pallas as pl

BS = 1024
NJ = 17
TP = 24
TOKN = 34
TOKP = 40
TCN = 2048
TD = 512
NBLK = 4

ENC_B = 32
VQR = 1024
DEC_B = 32

_F32 = jnp.float32


def _full_spec(x):
    nd = x.ndim
    return pl.BlockSpec(x.shape, lambda i, _nd=nd: (0,) * _nd)


def _ln(x, g, b):
    mu = jnp.mean(x, axis=-1, keepdims=True)
    m2 = jnp.mean(x * x, axis=-1, keepdims=True)
    var = m2 - mu * mu
    return (x - mu) * jax.lax.rsqrt(var + 1e-5) * g + b


def _gelu(x):
    return 0.5 * x * (1.0 + jax.lax.erf(x * (2.0 ** -0.5)))


def _dot2d(a, b, dt=_F32):
    return jax.lax.dot_general(a.astype(dt), b.astype(dt),
                               (((1,), (0,)), ((), ())),
                               preferred_element_type=_F32)


def _tok_dot(w, x, dt=_F32):
    bsz = x.shape[0]
    wb = jnp.broadcast_to(w[None].astype(dt), (bsz,) + w.shape)
    return jax.lax.dot_general(wb, x.astype(dt), (((1,), (1,)), ((0,), (0,))),
                               preferred_element_type=_F32)


def _ch_dot(x3, w, dt=_F32):
    b, t, d = x3.shape
    y2 = _dot2d(x3.reshape(b * t, d), w, dt)
    return y2.reshape(b, t, w.shape[1])


def _mixer(x, p, i, dt=_F32):
    (l1g, l1b, tw1, tb1, tw2, tb2, l2g, l2b, cw1, cb1, cw2, cb2) = p
    y = _ln(x, l1g[i], l1b[i])
    t = _tok_dot(tw1[i], y, dt) + tb1[i][None, :, None]
    t = _gelu(t)
    y = _tok_dot(tw2[i], t, dt) + tb2[i][None, :, None]
    z = _ln(x + y, l2g[i], l2b[i])
    z = _ch_dot(z, cw1[i], dt) + cb1[i][None, None, :]
    z = _gelu(z)
    z = _ch_dot(z, cw2[i], dt) + cb2[i][None, None, :]
    return x + y + z


def _enc_body(j_ref, inv_ref, sw_ref, sb_ref,
              l1g, l1b, tw1, tb1, tw2, tb2, l2g, l2b, cw1, cb1, cw2, cb2,
              elg, elb, tmw, tmb, few, feb, out_ref):
    j = j_ref[...]
    sw = sw_ref[...]
    x = (j[:, :, 0:1] * sw[0][None, None, :]
         + j[:, :, 1:2] * sw[1][None, None, :]
         + sb_ref[...][None, None, :])
    vis = j[:, :, 2:3] != 0.0
    x = jnp.where(vis, x, inv_ref[...])
    p = (l1g[...], l1b[...], tw1[...], tb1[...], tw2[...], tb2[...],
         l2g[...], l2b[...], cw1[...], cb1[...], cw2[...], cb2[...])
    for i in range(NBLK):
        x = _mixer(x, p, i)
    x = _ln(x, elg[...], elb[...])
    x = _tok_dot(tmw[...], x) + tmb[...][None, :, None]
    x = _ch_dot(x, few[...]) + feb[...][None, None, :]
    out_ref[...] = x


def _vq_body(f_ref, cb_ref, out_ref):
    f = f_ref[...] * -2.0
    cb = cb_ref[...]
    d = jax.lax.dot_general(f, cb, (((1,), (1,)), ((), ())),
                            preferred_element_type=_F32)
    ones = jnp.ones((8, TD), dtype=_F32)
    cn = jax.lax.dot_general(ones, cb * cb, (((1,), (1,)), ((), ())),
                             preferred_element_type=_F32)
    dist = cn[0:1, :] + d
    m = jnp.min(dist, axis=1, keepdims=True)
    it = jax.lax.broadcasted_iota(jnp.int32, dist.shape, 1)
    idx = jnp.min(jnp.where(dist == m, it, TCN), axis=1)
    out_ref[...] = idx.reshape(VQR // 128, 128)


def _dec_body(cls_ref, cb_ref, mk_ref, dtb_ref, dsw_ref, dsb_ref,
              l1g, l1b, tw1, tb1, tw2, tb2, l2g, l2b, cw1, cb1, cw2, cb2,
              dlg, dlb, rw_ref, rb_ref, out_ref):
    part = _dot2d(cls_ref[...], cb_ref[...], jnp.bfloat16)
    pt = _dot2d(mk_ref[...], part)
    x = pt.reshape(DEC_B, TP, TD) + dtb_ref[...][None, :, None]
    x = _ch_dot(x, dsw_ref[...]) + dsb_ref[...][None, None, :]
    p = (l1g[...], l1b[...], tw1[...], tb1[...], tw2[...], tb2[...],
         l2g[...], l2b[...], cw1[...], cb1[...], cw2[...], cb2[...])
    for i in range(NBLK):
        x = _mixer(x, p, i)
    x = _ln(x, dlg[...], dlb[...])
    r = _ch_dot(x, rw_ref[...]) + rb_ref[...][None, None, :]
    out_ref[...] = r


def _pad_axis(a, axis, n):
    pads = [(0, 0)] * a.ndim
    pads[axis] = (0, n - a.shape[axis])
    return jnp.pad(a, pads)


def kernel(joints, joints_feature, cls_logits, params):
    del joints_feature
    p = params
    enc, dec = p['enc'], p['dec']

    jp = _pad_axis(joints.astype(_F32), 1, TP)

    enc_args = [
        jp, p['invisible_token'], p['start_W'], p['start_b'],
        enc['ln1_g'], enc['ln1_b'],
        _pad_axis(enc['tok_W1'], 1, TP), enc['tok_b1'],
        _pad_axis(enc['tok_W2'], 2, TP), _pad_axis(enc['tok_b2'], 1, TP),
        enc['ln2_g'], enc['ln2_b'],
        enc['ch_W1'], enc['ch_b1'], enc['ch_W2'], enc['ch_b2'],
        p['enc_ln_g'], p['enc_ln_b'],
        _pad_axis(_pad_axis(p['tokmlp_W'], 0, TP), 1, TOKP),
        _pad_axis(p['tokmlp_b'], 0, TOKP),
        p['featemb_W'], p['featemb_b'],
    ]
    enc_specs = [pl.BlockSpec((ENC_B, TP, 3), lambda i: (i, 0, 0))]
    enc_specs += [_full_spec(a) for a in enc_args[1:]]
    feat = pl.pallas_call(
        _enc_body,
        grid=(BS // ENC_B,),
        in_specs=enc_specs,
        out_specs=pl.BlockSpec((ENC_B, TOKP, TD), lambda i: (i, 0, 0)),
        out_shape=jax.ShapeDtypeStruct((BS, TOKP, TD), _F32),
    )(*enc_args)

    cb = p['codebook']
    nrow = BS * TOKP
    idx_pad = pl.pallas_call(
        _vq_body,
        grid=(nrow // VQR,),
        in_specs=[pl.BlockSpec((VQR, TD), lambda i: (i, 0)),
                  _full_spec(cb)],
        out_specs=pl.BlockSpec((VQR // 128, 128), lambda i: (i, 0)),
        out_shape=jax.ShapeDtypeStruct((nrow // 128, 128), jnp.int32),
    )(feat.reshape(nrow, TD), cb)
    idx = idx_pad.reshape(BS, TOKP)[:, :TOKN].reshape(-1)

    wp = _pad_axis(p['dectok_W'], 1, TP)
    eye = jnp.eye(DEC_B, dtype=_F32)
    mk = (eye[:, None, :, None] * wp.T[None, :, None, :]).reshape(
        DEC_B * TP, DEC_B * TOKN)

    dec_args = [
        cls_logits, cb, mk, _pad_axis(p['dectok_b'], 0, TP),
        p['decstart_W'], p['decstart_b'],
        dec['ln1_g'], dec['ln1_b'],
        _pad_axis(dec['tok_W1'], 1, TP), dec['tok_b1'],
        _pad_axis(dec['tok_W2'], 2, TP), _pad_axis(dec['tok_b2'], 1, TP),
        dec['ln2_g'], dec['ln2_b'],
        dec['ch_W1'], dec['ch_b1'], dec['ch_W2'], dec['ch_b2'],
        p['dec_ln_g'], p['dec_ln_b'],
        p['rec_W'], p['rec_b'],
    ]
    dec_specs = [pl.BlockSpec((DEC_B * TOKN, TCN), lambda i: (i, 0))]
    dec_specs += [_full_spec(a) for a in dec_args[1:]]
    rec_pad = pl.pallas_call(
        _dec_body,
        grid=(BS // DEC_B,),
        in_specs=dec_specs,
        out_specs=pl.BlockSpec((DEC_B, TP, 2), lambda i: (i, 0, 0)),
        out_shape=jax.ShapeDtypeStruct((BS, TP, 2), _F32),
    )(*dec_args)
    rec = rec_pad[:, :NJ, :]
    return rec, idx
